# baseline (speedup 1.0000x reference)
import jax
import jax.numpy as jnp
from jax import lax
from jax.experimental import pallas as pl

N = 524288
D = 64
TAB = 64
BR = 4096


def _body(z_ref, stab_ref, htab_ref, x_ref, o_ref):
    zc = z_ref[...]
    lane = lax.broadcasted_iota(jnp.int32, (BR, TAB), 1)
    eq = zc == lane
    s = jnp.sum(jnp.where(eq, stab_ref[...], 0.0), axis=1, keepdims=True)
    h = jnp.sum(jnp.where(eq, htab_ref[...], 0.0), axis=1, keepdims=True)
    o_ref[...] = x_ref[...] * s + h


def kernel(input, z, scale_table, shift_table):
    zc = z.astype(jnp.int32).reshape(N, 1)
    stab = jnp.zeros((1, TAB), jnp.float32).at[0, :54].set(scale_table[:, 0])
    htab = jnp.zeros((1, TAB), jnp.float32).at[0, :54].set(shift_table[:, 0])
    grid = (N // BR,)
    return pl.pallas_call(
        _body,
        grid=grid,
        in_specs=[
            pl.BlockSpec((BR, 1), lambda i: (i, 0)),
            pl.BlockSpec((1, TAB), lambda i: (0, 0)),
            pl.BlockSpec((1, TAB), lambda i: (0, 0)),
            pl.BlockSpec((BR, D), lambda i: (i, 0)),
        ],
        out_specs=pl.BlockSpec((BR, D), lambda i: (i, 0)),
        out_shape=jax.ShapeDtypeStruct((N, D), jnp.float32),
    )(zc, stab, htab, input)
